# baseline (speedup 1.0000x reference)
import functools

import jax
import jax.numpy as jnp
from jax import lax
from jax.experimental import pallas as pl
from jax.experimental.pallas import tpu as pltpu
from jax.experimental.pallas import tpu_sc as plsc

N = 10000
E = 320000
EH = E // 2
D_IN = 128
D_NODE = 128
D_EATTR = 4
D_EDGE = 64
N_TYPES = 17

NC = 2
NS = 16
NW = NC * NS

C = 40
CPW = EH // C // NW
GRP = 5

NPAD = 10240
RPT = NPAD // NS

BN = 2000
BE = 5000

F32 = jnp.float32
HI = jax.lax.Precision.HIGHEST


def _dot(a, b):
    return jnp.dot(a, b, preferred_element_type=F32, precision=HI)


def _mesh():
    return plsc.VectorSubcoreMesh(core_axis_name="c", subcore_axis_name="s")


def _sc_gather(UV1, UV2, src, dst, half):
    @functools.partial(
        pl.kernel,
        out_type=[jax.ShapeDtypeStruct((EH, D_NODE), F32),
                  jax.ShapeDtypeStruct((EH, D_NODE), F32)],
        mesh=_mesh(),
        scratch_types=(
            [pltpu.VMEM((GRP * C,), jnp.int32), pltpu.VMEM((GRP * C,), jnp.int32)]
            + [pltpu.VMEM((C, D_NODE), F32) for _ in range(2 * GRP)]
            + [pltpu.SemaphoreType.DMA, pltpu.SemaphoreType.DMA,
               pltpu.SemaphoreType.DMA]
        ),
    )
    def k(u_hbm, v_hbm, src_hbm, dst_hbm, g1_hbm, g2_hbm, *rest):
        src_v, dst_v = rest[0], rest[1]
        bufs = rest[2:2 + 2 * GRP]
        isem, gsem, wsem = rest[2 + 2 * GRP:]
        wid = lax.axis_index("s") * NC + lax.axis_index("c")
        ebase = half * EH + wid * CPW * C

        def group(g, carry):
            e0 = ebase + g * (GRP * C)
            o0 = e0 - half * EH
            hi0 = pltpu.async_copy(src_hbm.at[pl.ds(e0, GRP * C)], src_v, isem)
            hi1 = pltpu.async_copy(dst_hbm.at[pl.ds(e0, GRP * C)], dst_v, isem)
            hi0.wait()
            hi1.wait()
            hs = []
            for b in range(GRP):
                iu = src_v.at[pl.ds(b * C, C)]
                iv = dst_v.at[pl.ds(b * C, C)]
                hs.append(pltpu.async_copy(u_hbm.at[iu], bufs[2 * b], gsem))
                hs.append(pltpu.async_copy(v_hbm.at[iv], bufs[2 * b + 1], gsem))
            for h in hs:
                h.wait()
            ws = []
            for b in range(GRP):
                row0 = o0 + b * C
                ws.append(pltpu.async_copy(bufs[2 * b], g1_hbm.at[pl.ds(row0, C)], wsem))
                ws.append(pltpu.async_copy(bufs[2 * b + 1], g2_hbm.at[pl.ds(row0, C)], wsem))
            for h in ws:
                h.wait()
            return carry

        lax.fori_loop(0, CPW // GRP, group, 0)

    return k(UV1, UV2, src, dst)


def _sc_scatter(e_feat, dst, zeros_init, half):
    @functools.partial(
        pl.kernel,
        out_type=jax.ShapeDtypeStruct((NC, NPAD, D_NODE), F32),
        mesh=_mesh(),
        scratch_types=(
            [pltpu.VMEM((C,), jnp.int32) for _ in range(GRP)]
            + [pltpu.VMEM((GRP * C, D_NODE), F32),
               pltpu.VMEM_SHARED((NPAD, D_NODE), F32),
               pltpu.SemaphoreType.DMA, pltpu.SemaphoreType.DMA]
        ),
    )
    def k(e_hbm, dst_hbm, z_hbm, out_hbm, *rest):
        idxbufs = rest[:GRP]
        dbuf, acc_sh = rest[GRP], rest[GRP + 1]
        isem, ssem = rest[GRP + 2], rest[GRP + 3]
        c = lax.axis_index("c")
        s = lax.axis_index("s")
        wid = s * NC + c
        obase = wid * CPW * C
        ibase = half * EH + obase
        pltpu.sync_copy(z_hbm.at[pl.ds(s * RPT, RPT)], acc_sh.at[pl.ds(s * RPT, RPT)])
        plsc.subcore_barrier()

        def group(g, carry):
            o0 = obase + g * (GRP * C)
            i0 = ibase + g * (GRP * C)
            his = [pltpu.async_copy(dst_hbm.at[pl.ds(i0 + b * C, C)], idxbufs[b], isem)
                   for b in range(GRP)]
            pltpu.sync_copy(e_hbm.at[pl.ds(o0, GRP * C)], dbuf)
            for h in his:
                h.wait()
            hs = []
            for b in range(GRP):
                hs.append(pltpu.async_copy(
                    dbuf.at[pl.ds(b * C, C)], acc_sh.at[idxbufs[b]], ssem,
                    add=True))
            for h in hs:
                h.wait()
            return carry

        lax.fori_loop(0, CPW // GRP, group, 0)
        plsc.subcore_barrier()
        pltpu.sync_copy(acc_sh.at[pl.ds(s * RPT, RPT)],
                        out_hbm.at[c, pl.ds(s * RPT, RPT)])

    return k(e_feat, dst, zeros_init)


def _full(shape):
    nd = len(shape)
    return pl.BlockSpec(shape, lambda i, _n=nd: (0,) * _n)


def _tc_embed(x, nt, Wst, bst, W1sd, W1ds):
    def body(x_ref, nt_ref, wst_ref, bst_ref, w1s_ref, w1d_ref,
             nf_ref, u_ref, v_ref):
        xb = x_ref[...]
        ntb = nt_ref[...]
        acc = jnp.zeros((BN, D_NODE), F32)
        for t in range(N_TYPES):
            yt = _dot(xb, wst_ref[t]) + bst_ref[t][None]
            acc = acc + jnp.where(ntb == t, yt, 0.0)
        nf_ref[...] = acc
        u_ref[...] = _dot(acc, w1s_ref[...])
        v_ref[...] = _dot(acc, w1d_ref[...])

    return pl.pallas_call(
        body,
        grid=(N // BN,),
        in_specs=[
            pl.BlockSpec((BN, D_IN), lambda i: (i, 0)),
            pl.BlockSpec((BN, 1), lambda i: (i, 0)),
            _full((N_TYPES, D_IN, D_NODE)),
            _full((N_TYPES, D_NODE)),
            _full((D_NODE, D_NODE)),
            _full((D_NODE, D_NODE)),
        ],
        out_specs=[
            pl.BlockSpec((BN, D_NODE), lambda i: (i, 0)),
            pl.BlockSpec((BN, D_NODE), lambda i: (i, 0)),
            pl.BlockSpec((BN, D_NODE), lambda i: (i, 0)),
        ],
        out_shape=[
            jax.ShapeDtypeStruct((N, D_NODE), F32),
            jax.ShapeDtypeStruct((N, D_NODE), F32),
            jax.ShapeDtypeStruct((N, D_NODE), F32),
        ],
    )(x, nt, Wst, bst, W1sd, W1ds)


def _tc_edge1(G1, G2, ea, We, be, W1e_p, b1_p, W2_pp, b2_p):
    def body(g1_ref, g2_ref, ea_ref, we_ref, be_ref, w1e_ref, b1_ref,
             w2_ref, b2_ref, e_ref):
        wfold = _dot(we_ref[...], w1e_ref[...])
        bfold = _dot(be_ref[...], w1e_ref[...])
        p = _dot(ea_ref[...], wfold) + bfold
        h = jnp.maximum(g1_ref[...] + g2_ref[...] + p + b1_ref[...], 0.0)
        e_ref[...] = _dot(h, w2_ref[...]) + b2_ref[...]

    return pl.pallas_call(
        body,
        grid=(EH // BE,),
        in_specs=[
            pl.BlockSpec((BE, D_NODE), lambda i: (i, 0)),
            pl.BlockSpec((BE, D_NODE), lambda i: (i, 0)),
            pl.BlockSpec((BE, D_EATTR), lambda i: (i, 0)),
            _full((D_EATTR, D_EDGE)),
            _full((1, D_EDGE)),
            _full((D_EDGE, D_NODE)),
            _full((1, D_NODE)),
            _full((D_NODE, D_NODE)),
            _full((1, D_NODE)),
        ],
        out_specs=pl.BlockSpec((BE, D_NODE), lambda i: (i, 0)),
        out_shape=jax.ShapeDtypeStruct((EH, D_NODE), F32),
    )(G1, G2, ea, We, be, W1e_p, b1_p, W2_pp, b2_p)


def _tc_edge2(G1, G2, ep, W1e_pp, b1_p, W2_pp, b2_p, Wec1_p, bec1, Wec2, bec2):
    def body(g1_ref, g2_ref, ep_ref, w1e_ref, b1_ref, w2_ref, b2_ref,
             wec1_ref, bec1_ref, wec2_ref, bec2_ref, e_ref, pe_ref):
        p = _dot(ep_ref[...], w1e_ref[...])
        h = jnp.maximum(g1_ref[...] + g2_ref[...] + p + b1_ref[...], 0.0)
        e_new = _dot(h, w2_ref[...]) + b2_ref[...]
        e_ref[...] = e_new
        t = jnp.maximum(_dot(e_new, wec1_ref[...]) + bec1_ref[...], 0.0)
        pe_ref[...] = _dot(t, wec2_ref[...]) + bec2_ref[...]

    return pl.pallas_call(
        body,
        grid=(EH // BE,),
        in_specs=[
            pl.BlockSpec((BE, D_NODE), lambda i: (i, 0)),
            pl.BlockSpec((BE, D_NODE), lambda i: (i, 0)),
            pl.BlockSpec((BE, D_NODE), lambda i: (i, 0)),
            _full((D_NODE, D_NODE)),
            _full((1, D_NODE)),
            _full((D_NODE, D_NODE)),
            _full((1, D_NODE)),
            _full((D_NODE, 32)),
            _full((1, 32)),
            _full((32, 1)),
            _full((1, 1)),
        ],
        out_specs=[
            pl.BlockSpec((BE, D_NODE), lambda i: (i, 0)),
            pl.BlockSpec((BE, 1), lambda i: (i, 0)),
        ],
        out_shape=[
            jax.ShapeDtypeStruct((EH, D_NODE), F32),
            jax.ShapeDtypeStruct((EH, 1), F32),
        ],
    )(G1, G2, ep, W1e_pp, b1_p, W2_pp, b2_p, Wec1_p, bec1, Wec2, bec2)


def _tc_edge3(G1, G2, ep, W1e_pp, b1_p, W2_pp, b2_p):
    def body(g1_ref, g2_ref, ep_ref, w1e_ref, b1_ref, w2_ref, b2_ref,
             e_ref, e64_ref):
        p = _dot(ep_ref[...], w1e_ref[...])
        h = jnp.maximum(g1_ref[...] + g2_ref[...] + p + b1_ref[...], 0.0)
        e_new = _dot(h, w2_ref[...]) + b2_ref[...]
        e_ref[...] = e_new
        e64_ref[...] = e_new[:, :D_EDGE]

    return pl.pallas_call(
        body,
        grid=(EH // BE,),
        in_specs=[
            pl.BlockSpec((BE, D_NODE), lambda i: (i, 0)),
            pl.BlockSpec((BE, D_NODE), lambda i: (i, 0)),
            pl.BlockSpec((BE, D_NODE), lambda i: (i, 0)),
            _full((D_NODE, D_NODE)),
            _full((1, D_NODE)),
            _full((D_NODE, D_NODE)),
            _full((1, D_NODE)),
        ],
        out_specs=[
            pl.BlockSpec((BE, D_NODE), lambda i: (i, 0)),
            pl.BlockSpec((BE, D_EDGE), lambda i: (i, 0)),
        ],
        out_shape=[
            jax.ShapeDtypeStruct((EH, D_NODE), F32),
            jax.ShapeDtypeStruct((EH, D_EDGE), F32),
        ],
    )(G1, G2, ep, W1e_pp, b1_p, W2_pp, b2_p)


def _tc_node(nf, aggs, WnA, WnB_p, bn, W1sd, W1ds):
    def body(nf_ref, a0_ref, a1_ref, a2_ref, a3_ref,
             wna_ref, wnb_ref, bn_ref, w1s_ref, w1d_ref,
             o_ref, u_ref, v_ref):
        agg = a0_ref[...] + a1_ref[...] + a2_ref[...] + a3_ref[...]
        nn = _dot(nf_ref[...], wna_ref[...]) + _dot(agg, wnb_ref[...]) + bn_ref[...]
        o_ref[...] = nn
        u_ref[...] = _dot(nn, w1s_ref[...])
        v_ref[...] = _dot(nn, w1d_ref[...])

    aspec = pl.BlockSpec((BN, D_NODE), lambda i: (i, 0))
    return pl.pallas_call(
        body,
        grid=(N // BN,),
        in_specs=[aspec, aspec, aspec, aspec, aspec,
                  _full((D_NODE, D_NODE)),
                  _full((D_NODE, D_NODE)),
                  _full((1, D_NODE)),
                  _full((D_NODE, D_NODE)),
                  _full((D_NODE, D_NODE))],
        out_specs=[aspec, aspec, aspec],
        out_shape=[
            jax.ShapeDtypeStruct((N, D_NODE), F32),
            jax.ShapeDtypeStruct((N, D_NODE), F32),
            jax.ShapeDtypeStruct((N, D_NODE), F32),
        ],
    )(nf, aggs[0], aggs[1], aggs[2], aggs[3], WnA, WnB_p, bn, W1sd, W1ds)


def _tc_node3(nf, aggs, WnA, WnB_p, bn, Wnc1, bnc1, Wnc2, bnc2, Wc1, bc1, Wc2, bc2):
    def body(nf_ref, a0_ref, a1_ref, a2_ref, a3_ref, wna_ref, wnb_ref, bn_ref,
             wnc1_ref, bnc1_ref, wnc2_ref, bnc2_ref,
             wc1_ref, bc1_ref, wc2_ref, bc2_ref,
             o_ref, pn_ref, pc_ref):
        agg = a0_ref[...] + a1_ref[...] + a2_ref[...] + a3_ref[...]
        nn = _dot(nf_ref[...], wna_ref[...]) + _dot(agg, wnb_ref[...]) + bn_ref[...]
        o_ref[...] = nn
        t1 = jnp.maximum(_dot(nn, wnc1_ref[...]) + bnc1_ref[...], 0.0)
        pn_ref[...] = _dot(t1, wnc2_ref[...]) + bnc2_ref[...]
        t2 = jnp.maximum(_dot(nn, wc1_ref[...]) + bc1_ref[...], 0.0)
        pc_ref[...] = _dot(t2, wc2_ref[...]) + bc2_ref[...]

    aspec = pl.BlockSpec((BN, D_NODE), lambda i: (i, 0))
    return pl.pallas_call(
        body,
        grid=(N // BN,),
        in_specs=[aspec, aspec, aspec, aspec, aspec,
                  _full((D_NODE, D_NODE)),
                  _full((D_NODE, D_NODE)),
                  _full((1, D_NODE)),
                  _full((D_NODE, 32)),
                  _full((1, 32)),
                  _full((32, 1)),
                  _full((1, 1)),
                  _full((D_NODE, 32)),
                  _full((1, 32)),
                  _full((32, 8)),
                  _full((1, 8))],
        out_specs=[aspec,
                   pl.BlockSpec((BN, 1), lambda i: (i, 0)),
                   pl.BlockSpec((BN, 8), lambda i: (i, 0))],
        out_shape=[
            jax.ShapeDtypeStruct((N, D_NODE), F32),
            jax.ShapeDtypeStruct((N, 1), F32),
            jax.ShapeDtypeStruct((N, 8), F32),
        ],
    )(nf, aggs[0], aggs[1], aggs[2], aggs[3], WnA, WnB_p, bn,
      Wnc1, bnc1, Wnc2, bnc2, Wc1, bc1, Wc2, bc2)


def kernel(x, edge_attr, edge_index, node_types, params):
    src = edge_index[0]
    dst = edge_index[1]
    nt = node_types.reshape(N, 1)

    Wst = jnp.stack([p[0][0] for p in params['node_mlps']])
    bst = jnp.stack([p[0][1] for p in params['node_mlps']])
    We, be = params['edge_emb'][0]
    (W1, b1), (W2, b2) = params['mp_edge']
    Wn, bn = params['mp_node'][0]
    (Wec1, bec1), (Wec2, bec2) = params['edge_cls']
    (Wnc1, bnc1), (Wnc2, bnc2) = params['node_cls']
    (Wc1, bc1), (Wc2, bc2) = params['cls']

    W1s, W1d, W1e = W1[:D_NODE], W1[D_NODE:2 * D_NODE], W1[2 * D_NODE:]
    WnA, WnB = Wn[:D_NODE], Wn[D_NODE:]
    r1 = lambda v: v.reshape(1, -1)

    zc = jnp.zeros((D_EDGE, D_EDGE), F32)
    zr = jnp.zeros((D_EDGE,), F32)
    W1sd = jnp.concatenate([W1s, W1d], axis=1)
    W1ds = jnp.concatenate([W1d, W1s], axis=1)
    W1e_p = jnp.concatenate([W1e, zc], axis=1)
    W1e_pp = jnp.concatenate([W1e_p, jnp.zeros((D_EDGE, D_NODE), F32)], axis=0)
    W2_pp = jnp.concatenate(
        [jnp.concatenate([W2, zc], axis=1),
         jnp.zeros((D_EDGE, D_NODE), F32)], axis=0)
    b1_p = jnp.concatenate([b1, zr])
    b2_p = jnp.concatenate([b2, zr])
    WnB_p = jnp.concatenate([WnB, jnp.zeros((D_EDGE, D_NODE), F32)], axis=0)
    Wec1_p = jnp.concatenate([Wec1, jnp.zeros((D_EDGE, 32), F32)], axis=0)

    zeros_init = jnp.zeros((NPAD, D_NODE), F32)
    ea0, ea1 = edge_attr[:EH], edge_attr[EH:]

    nf, U, V = _tc_embed(x, nt, Wst, bst, W1sd, W1ds)

    def layer(U, V, edge_fn, edge_args_a, edge_args_b):
        g = [_sc_gather(U, V, src, dst, h) for h in (0, 1)]
        ea_out = edge_fn(g[0][0], g[0][1], *edge_args_a)
        eb_out = edge_fn(g[1][0], g[1][1], *edge_args_b)
        return ea_out, eb_out

    (e1a,), (e1b,) = layer(
        U, V, lambda *a: (_tc_edge1(*a),),
        (ea0, We, r1(be), W1e_p, r1(b1_p), W2_pp, r1(b2_p)),
        (ea1, We, r1(be), W1e_p, r1(b1_p), W2_pp, r1(b2_p)))
    p0 = _sc_scatter(e1a, dst, zeros_init, 0)
    p1 = _sc_scatter(e1b, dst, zeros_init, 1)
    aggs = (p0[0, :N], p0[1, :N], p1[0, :N], p1[1, :N])
    nf, U, V = _tc_node(nf, aggs, WnA, WnB_p, r1(bn), W1sd, W1ds)

    wargs2 = (W1e_pp, r1(b1_p), W2_pp, r1(b2_p), Wec1_p, r1(bec1), Wec2, r1(bec2))
    (e2a, pea), (e2b, peb) = layer(
        U, V, _tc_edge2, (e1a,) + wargs2, (e1b,) + wargs2)
    p0 = _sc_scatter(e2a, dst, zeros_init, 0)
    p1 = _sc_scatter(e2b, dst, zeros_init, 1)
    aggs = (p0[0, :N], p0[1, :N], p1[0, :N], p1[1, :N])
    nf, U, V = _tc_node(nf, aggs, WnA, WnB_p, r1(bn), W1sd, W1ds)

    wargs3 = (W1e_pp, r1(b1_p), W2_pp, r1(b2_p))
    (e3a, e3a64), (e3b, e3b64) = layer(
        U, V, _tc_edge3, (e2a,) + wargs3, (e2b,) + wargs3)
    p0 = _sc_scatter(e3a, dst, zeros_init, 0)
    p1 = _sc_scatter(e3b, dst, zeros_init, 1)
    aggs = (p0[0, :N], p0[1, :N], p1[0, :N], p1[1, :N])
    nf3, pn, pc = _tc_node3(nf, aggs, WnA, WnB_p, r1(bn),
                            Wnc1, r1(bnc1), Wnc2, r1(bnc2),
                            Wc1, r1(bc1), Wc2, r1(bc2))

    pe = jnp.concatenate([pea, peb], axis=0)
    e3_64 = jnp.concatenate([e3a64, e3b64], axis=0)
    return (pe.reshape(E), pn.reshape(N), pc, nf3, e3_64)


# speedup vs baseline: 2.1302x; 2.1302x over previous
import functools

import jax
import jax.numpy as jnp
from jax import lax
from jax.experimental import pallas as pl
from jax.experimental.pallas import tpu as pltpu
from jax.experimental.pallas import tpu_sc as plsc

N = 10000
E = 320000
EH = E // 2
D_IN = 128
D_NODE = 128
D_EATTR = 4
D_EDGE = 64
N_TYPES = 17

NC = 2
NS = 16
NW = NC * NS

C = 40
CPW = EH // C // NW
GRP = 5

NPAD = 10240
RPT = NPAD // NS

BN = 2000
BE = 5000

F32 = jnp.float32
HI = jax.lax.Precision.DEFAULT


def _dot(a, b):
    return jnp.dot(a, b, preferred_element_type=F32, precision=HI)


def _mesh():
    return plsc.VectorSubcoreMesh(core_axis_name="c", subcore_axis_name="s")


def _sc_gather(UV1, UV2, src, dst, half):
    @functools.partial(
        pl.kernel,
        out_type=[jax.ShapeDtypeStruct((EH, D_NODE), F32),
                  jax.ShapeDtypeStruct((EH, D_NODE), F32)],
        mesh=_mesh(),
        scratch_types=(
            [pltpu.VMEM((GRP * C,), jnp.int32), pltpu.VMEM((GRP * C,), jnp.int32)]
            + [pltpu.VMEM((C, D_NODE), F32) for _ in range(2 * GRP)]
            + [pltpu.SemaphoreType.DMA, pltpu.SemaphoreType.DMA,
               pltpu.SemaphoreType.DMA]
        ),
    )
    def k(u_hbm, v_hbm, src_hbm, dst_hbm, g1_hbm, g2_hbm, *rest):
        src_v, dst_v = rest[0], rest[1]
        bufs = rest[2:2 + 2 * GRP]
        isem, gsem, wsem = rest[2 + 2 * GRP:]
        wid = lax.axis_index("s") * NC + lax.axis_index("c")
        ebase = half * EH + wid * CPW * C

        def group(g, carry):
            e0 = ebase + g * (GRP * C)
            o0 = e0 - half * EH
            hi0 = pltpu.async_copy(src_hbm.at[pl.ds(e0, GRP * C)], src_v, isem)
            hi1 = pltpu.async_copy(dst_hbm.at[pl.ds(e0, GRP * C)], dst_v, isem)
            hi0.wait()
            hi1.wait()
            hs = []
            for b in range(GRP):
                iu = src_v.at[pl.ds(b * C, C)]
                iv = dst_v.at[pl.ds(b * C, C)]
                hs.append(pltpu.async_copy(u_hbm.at[iu], bufs[2 * b], gsem))
                hs.append(pltpu.async_copy(v_hbm.at[iv], bufs[2 * b + 1], gsem))
            for h in hs:
                h.wait()
            ws = []
            for b in range(GRP):
                row0 = o0 + b * C
                ws.append(pltpu.async_copy(bufs[2 * b], g1_hbm.at[pl.ds(row0, C)], wsem))
                ws.append(pltpu.async_copy(bufs[2 * b + 1], g2_hbm.at[pl.ds(row0, C)], wsem))
            for h in ws:
                h.wait()
            return carry

        lax.fori_loop(0, CPW // GRP, group, 0)

    return k(UV1, UV2, src, dst)


def _sc_scatter(e_feat, dst, zeros_init, half):
    @functools.partial(
        pl.kernel,
        out_type=jax.ShapeDtypeStruct((NC, NPAD, D_NODE), F32),
        mesh=_mesh(),
        scratch_types=(
            [pltpu.VMEM((C,), jnp.int32) for _ in range(GRP)]
            + [pltpu.VMEM((GRP * C, D_NODE), F32),
               pltpu.VMEM_SHARED((NPAD, D_NODE), F32),
               pltpu.SemaphoreType.DMA, pltpu.SemaphoreType.DMA]
        ),
    )
    def k(e_hbm, dst_hbm, z_hbm, out_hbm, *rest):
        idxbufs = rest[:GRP]
        dbuf, acc_sh = rest[GRP], rest[GRP + 1]
        isem, ssem = rest[GRP + 2], rest[GRP + 3]
        c = lax.axis_index("c")
        s = lax.axis_index("s")
        wid = s * NC + c
        obase = wid * CPW * C
        ibase = half * EH + obase
        pltpu.sync_copy(z_hbm.at[pl.ds(s * RPT, RPT)], acc_sh.at[pl.ds(s * RPT, RPT)])
        plsc.subcore_barrier()

        def group(g, carry):
            o0 = obase + g * (GRP * C)
            i0 = ibase + g * (GRP * C)
            his = [pltpu.async_copy(dst_hbm.at[pl.ds(i0 + b * C, C)], idxbufs[b], isem)
                   for b in range(GRP)]
            pltpu.sync_copy(e_hbm.at[pl.ds(o0, GRP * C)], dbuf)
            for h in his:
                h.wait()
            hs = []
            for b in range(GRP):
                hs.append(pltpu.async_copy(
                    dbuf.at[pl.ds(b * C, C)], acc_sh.at[idxbufs[b]], ssem,
                    add=True))
            for h in hs:
                h.wait()
            return carry

        lax.fori_loop(0, CPW // GRP, group, 0)
        plsc.subcore_barrier()
        pltpu.sync_copy(acc_sh.at[pl.ds(s * RPT, RPT)],
                        out_hbm.at[c, pl.ds(s * RPT, RPT)])

    return k(e_feat, dst, zeros_init)


def _full(shape):
    nd = len(shape)
    return pl.BlockSpec(shape, lambda i, _n=nd: (0,) * _n)


def _tc_embed(x, nt, Wst, bst, W1sd, W1ds):
    def body(x_ref, nt_ref, wst_ref, bst_ref, w1s_ref, w1d_ref,
             nf_ref, u_ref, v_ref):
        xb = x_ref[...]
        ntb = nt_ref[...]
        acc = jnp.zeros((BN, D_NODE), F32)
        for t in range(N_TYPES):
            yt = _dot(xb, wst_ref[t]) + bst_ref[t][None]
            acc = acc + jnp.where(ntb == t, yt, 0.0)
        nf_ref[...] = acc
        u_ref[...] = _dot(acc, w1s_ref[...])
        v_ref[...] = _dot(acc, w1d_ref[...])

    return pl.pallas_call(
        body,
        grid=(N // BN,),
        in_specs=[
            pl.BlockSpec((BN, D_IN), lambda i: (i, 0)),
            pl.BlockSpec((BN, 1), lambda i: (i, 0)),
            _full((N_TYPES, D_IN, D_NODE)),
            _full((N_TYPES, D_NODE)),
            _full((D_NODE, D_NODE)),
            _full((D_NODE, D_NODE)),
        ],
        out_specs=[
            pl.BlockSpec((BN, D_NODE), lambda i: (i, 0)),
            pl.BlockSpec((BN, D_NODE), lambda i: (i, 0)),
            pl.BlockSpec((BN, D_NODE), lambda i: (i, 0)),
        ],
        out_shape=[
            jax.ShapeDtypeStruct((N, D_NODE), F32),
            jax.ShapeDtypeStruct((N, D_NODE), F32),
            jax.ShapeDtypeStruct((N, D_NODE), F32),
        ],
    )(x, nt, Wst, bst, W1sd, W1ds)


def _tc_edge1(G1, G2, ea, We, be, W1e_p, b1_p, W2_pp, b2_p):
    def body(g1_ref, g2_ref, ea_ref, we_ref, be_ref, w1e_ref, b1_ref,
             w2_ref, b2_ref, e_ref):
        wfold = _dot(we_ref[...], w1e_ref[...])
        bfold = _dot(be_ref[...], w1e_ref[...])
        p = _dot(ea_ref[...], wfold) + bfold
        h = jnp.maximum(g1_ref[...] + g2_ref[...] + p + b1_ref[...], 0.0)
        e_ref[...] = _dot(h, w2_ref[...]) + b2_ref[...]

    return pl.pallas_call(
        body,
        grid=(EH // BE,),
        in_specs=[
            pl.BlockSpec((BE, D_NODE), lambda i: (i, 0)),
            pl.BlockSpec((BE, D_NODE), lambda i: (i, 0)),
            pl.BlockSpec((BE, D_EATTR), lambda i: (i, 0)),
            _full((D_EATTR, D_EDGE)),
            _full((1, D_EDGE)),
            _full((D_EDGE, D_NODE)),
            _full((1, D_NODE)),
            _full((D_NODE, D_NODE)),
            _full((1, D_NODE)),
        ],
        out_specs=pl.BlockSpec((BE, D_NODE), lambda i: (i, 0)),
        out_shape=jax.ShapeDtypeStruct((EH, D_NODE), F32),
    )(G1, G2, ea, We, be, W1e_p, b1_p, W2_pp, b2_p)


def _tc_edge2(G1, G2, ep, W1e_pp, b1_p, W2_pp, b2_p, Wec1_p, bec1, Wec2, bec2):
    def body(g1_ref, g2_ref, ep_ref, w1e_ref, b1_ref, w2_ref, b2_ref,
             wec1_ref, bec1_ref, wec2_ref, bec2_ref, e_ref, pe_ref):
        p = _dot(ep_ref[...], w1e_ref[...])
        h = jnp.maximum(g1_ref[...] + g2_ref[...] + p + b1_ref[...], 0.0)
        e_new = _dot(h, w2_ref[...]) + b2_ref[...]
        e_ref[...] = e_new
        t = jnp.maximum(_dot(e_new, wec1_ref[...]) + bec1_ref[...], 0.0)
        pe_ref[...] = _dot(t, wec2_ref[...]) + bec2_ref[...]

    return pl.pallas_call(
        body,
        grid=(EH // BE,),
        in_specs=[
            pl.BlockSpec((BE, D_NODE), lambda i: (i, 0)),
            pl.BlockSpec((BE, D_NODE), lambda i: (i, 0)),
            pl.BlockSpec((BE, D_NODE), lambda i: (i, 0)),
            _full((D_NODE, D_NODE)),
            _full((1, D_NODE)),
            _full((D_NODE, D_NODE)),
            _full((1, D_NODE)),
            _full((D_NODE, 32)),
            _full((1, 32)),
            _full((32, 1)),
            _full((1, 1)),
        ],
        out_specs=[
            pl.BlockSpec((BE, D_NODE), lambda i: (i, 0)),
            pl.BlockSpec((BE, 1), lambda i: (i, 0)),
        ],
        out_shape=[
            jax.ShapeDtypeStruct((EH, D_NODE), F32),
            jax.ShapeDtypeStruct((EH, 1), F32),
        ],
    )(G1, G2, ep, W1e_pp, b1_p, W2_pp, b2_p, Wec1_p, bec1, Wec2, bec2)


def _tc_edge3(G1, G2, ep, W1e_pp, b1_p, W2_pp, b2_p):
    def body(g1_ref, g2_ref, ep_ref, w1e_ref, b1_ref, w2_ref, b2_ref,
             e_ref, e64_ref):
        p = _dot(ep_ref[...], w1e_ref[...])
        h = jnp.maximum(g1_ref[...] + g2_ref[...] + p + b1_ref[...], 0.0)
        e_new = _dot(h, w2_ref[...]) + b2_ref[...]
        e_ref[...] = e_new
        e64_ref[...] = e_new[:, :D_EDGE]

    return pl.pallas_call(
        body,
        grid=(EH // BE,),
        in_specs=[
            pl.BlockSpec((BE, D_NODE), lambda i: (i, 0)),
            pl.BlockSpec((BE, D_NODE), lambda i: (i, 0)),
            pl.BlockSpec((BE, D_NODE), lambda i: (i, 0)),
            _full((D_NODE, D_NODE)),
            _full((1, D_NODE)),
            _full((D_NODE, D_NODE)),
            _full((1, D_NODE)),
        ],
        out_specs=[
            pl.BlockSpec((BE, D_NODE), lambda i: (i, 0)),
            pl.BlockSpec((BE, D_EDGE), lambda i: (i, 0)),
        ],
        out_shape=[
            jax.ShapeDtypeStruct((EH, D_NODE), F32),
            jax.ShapeDtypeStruct((EH, D_EDGE), F32),
        ],
    )(G1, G2, ep, W1e_pp, b1_p, W2_pp, b2_p)


def _tc_node(nf, aggs, WnA, WnB_p, bn, W1sd, W1ds):
    def body(nf_ref, a0_ref, a1_ref, a2_ref, a3_ref,
             wna_ref, wnb_ref, bn_ref, w1s_ref, w1d_ref,
             o_ref, u_ref, v_ref):
        agg = a0_ref[...] + a1_ref[...] + a2_ref[...] + a3_ref[...]
        nn = _dot(nf_ref[...], wna_ref[...]) + _dot(agg, wnb_ref[...]) + bn_ref[...]
        o_ref[...] = nn
        u_ref[...] = _dot(nn, w1s_ref[...])
        v_ref[...] = _dot(nn, w1d_ref[...])

    aspec = pl.BlockSpec((BN, D_NODE), lambda i: (i, 0))
    return pl.pallas_call(
        body,
        grid=(N // BN,),
        in_specs=[aspec, aspec, aspec, aspec, aspec,
                  _full((D_NODE, D_NODE)),
                  _full((D_NODE, D_NODE)),
                  _full((1, D_NODE)),
                  _full((D_NODE, D_NODE)),
                  _full((D_NODE, D_NODE))],
        out_specs=[aspec, aspec, aspec],
        out_shape=[
            jax.ShapeDtypeStruct((N, D_NODE), F32),
            jax.ShapeDtypeStruct((N, D_NODE), F32),
            jax.ShapeDtypeStruct((N, D_NODE), F32),
        ],
    )(nf, aggs[0], aggs[1], aggs[2], aggs[3], WnA, WnB_p, bn, W1sd, W1ds)


def _tc_node3(nf, aggs, WnA, WnB_p, bn, Wnc1, bnc1, Wnc2, bnc2, Wc1, bc1, Wc2, bc2):
    def body(nf_ref, a0_ref, a1_ref, a2_ref, a3_ref, wna_ref, wnb_ref, bn_ref,
             wnc1_ref, bnc1_ref, wnc2_ref, bnc2_ref,
             wc1_ref, bc1_ref, wc2_ref, bc2_ref,
             o_ref, pn_ref, pc_ref):
        agg = a0_ref[...] + a1_ref[...] + a2_ref[...] + a3_ref[...]
        nn = _dot(nf_ref[...], wna_ref[...]) + _dot(agg, wnb_ref[...]) + bn_ref[...]
        o_ref[...] = nn
        t1 = jnp.maximum(_dot(nn, wnc1_ref[...]) + bnc1_ref[...], 0.0)
        pn_ref[...] = _dot(t1, wnc2_ref[...]) + bnc2_ref[...]
        t2 = jnp.maximum(_dot(nn, wc1_ref[...]) + bc1_ref[...], 0.0)
        pc_ref[...] = _dot(t2, wc2_ref[...]) + bc2_ref[...]

    aspec = pl.BlockSpec((BN, D_NODE), lambda i: (i, 0))
    return pl.pallas_call(
        body,
        grid=(N // BN,),
        in_specs=[aspec, aspec, aspec, aspec, aspec,
                  _full((D_NODE, D_NODE)),
                  _full((D_NODE, D_NODE)),
                  _full((1, D_NODE)),
                  _full((D_NODE, 32)),
                  _full((1, 32)),
                  _full((32, 1)),
                  _full((1, 1)),
                  _full((D_NODE, 32)),
                  _full((1, 32)),
                  _full((32, 8)),
                  _full((1, 8))],
        out_specs=[aspec,
                   pl.BlockSpec((BN, 1), lambda i: (i, 0)),
                   pl.BlockSpec((BN, 8), lambda i: (i, 0))],
        out_shape=[
            jax.ShapeDtypeStruct((N, D_NODE), F32),
            jax.ShapeDtypeStruct((N, 1), F32),
            jax.ShapeDtypeStruct((N, 8), F32),
        ],
    )(nf, aggs[0], aggs[1], aggs[2], aggs[3], WnA, WnB_p, bn,
      Wnc1, bnc1, Wnc2, bnc2, Wc1, bc1, Wc2, bc2)


def kernel(x, edge_attr, edge_index, node_types, params):
    src = edge_index[0]
    dst = edge_index[1]
    nt = node_types.reshape(N, 1)

    Wst = jnp.stack([p[0][0] for p in params['node_mlps']])
    bst = jnp.stack([p[0][1] for p in params['node_mlps']])
    We, be = params['edge_emb'][0]
    (W1, b1), (W2, b2) = params['mp_edge']
    Wn, bn = params['mp_node'][0]
    (Wec1, bec1), (Wec2, bec2) = params['edge_cls']
    (Wnc1, bnc1), (Wnc2, bnc2) = params['node_cls']
    (Wc1, bc1), (Wc2, bc2) = params['cls']

    W1s, W1d, W1e = W1[:D_NODE], W1[D_NODE:2 * D_NODE], W1[2 * D_NODE:]
    WnA, WnB = Wn[:D_NODE], Wn[D_NODE:]
    r1 = lambda v: v.reshape(1, -1)

    zc = jnp.zeros((D_EDGE, D_EDGE), F32)
    zr = jnp.zeros((D_EDGE,), F32)
    W1sd = jnp.concatenate([W1s, W1d], axis=1)
    W1ds = jnp.concatenate([W1d, W1s], axis=1)
    W1e_p = jnp.concatenate([W1e, zc], axis=1)
    W1e_pp = jnp.concatenate([W1e_p, jnp.zeros((D_EDGE, D_NODE), F32)], axis=0)
    W2_pp = jnp.concatenate(
        [jnp.concatenate([W2, zc], axis=1),
         jnp.zeros((D_EDGE, D_NODE), F32)], axis=0)
    b1_p = jnp.concatenate([b1, zr])
    b2_p = jnp.concatenate([b2, zr])
    WnB_p = jnp.concatenate([WnB, jnp.zeros((D_EDGE, D_NODE), F32)], axis=0)
    Wec1_p = jnp.concatenate([Wec1, jnp.zeros((D_EDGE, 32), F32)], axis=0)

    zeros_init = jnp.zeros((NPAD, D_NODE), F32)
    ea0, ea1 = edge_attr[:EH], edge_attr[EH:]

    nf, U, V = _tc_embed(x, nt, Wst, bst, W1sd, W1ds)

    def layer(U, V, edge_fn, edge_args_a, edge_args_b):
        g = [_sc_gather(U, V, src, dst, h) for h in (0, 1)]
        ea_out = edge_fn(g[0][0], g[0][1], *edge_args_a)
        eb_out = edge_fn(g[1][0], g[1][1], *edge_args_b)
        return ea_out, eb_out

    (e1a,), (e1b,) = layer(
        U, V, lambda *a: (_tc_edge1(*a),),
        (ea0, We, r1(be), W1e_p, r1(b1_p), W2_pp, r1(b2_p)),
        (ea1, We, r1(be), W1e_p, r1(b1_p), W2_pp, r1(b2_p)))
    p0 = _sc_scatter(e1a, dst, zeros_init, 0)
    p1 = _sc_scatter(e1b, dst, zeros_init, 1)
    aggs = (p0[0, :N], p0[1, :N], p1[0, :N], p1[1, :N])
    nf, U, V = _tc_node(nf, aggs, WnA, WnB_p, r1(bn), W1sd, W1ds)

    wargs2 = (W1e_pp, r1(b1_p), W2_pp, r1(b2_p), Wec1_p, r1(bec1), Wec2, r1(bec2))
    (e2a, pea), (e2b, peb) = layer(
        U, V, _tc_edge2, (e1a,) + wargs2, (e1b,) + wargs2)
    p0 = _sc_scatter(e2a, dst, zeros_init, 0)
    p1 = _sc_scatter(e2b, dst, zeros_init, 1)
    aggs = (p0[0, :N], p0[1, :N], p1[0, :N], p1[1, :N])
    nf, U, V = _tc_node(nf, aggs, WnA, WnB_p, r1(bn), W1sd, W1ds)

    wargs3 = (W1e_pp, r1(b1_p), W2_pp, r1(b2_p))
    (e3a, e3a64), (e3b, e3b64) = layer(
        U, V, _tc_edge3, (e2a,) + wargs3, (e2b,) + wargs3)
    p0 = _sc_scatter(e3a, dst, zeros_init, 0)
    p1 = _sc_scatter(e3b, dst, zeros_init, 1)
    aggs = (p0[0, :N], p0[1, :N], p1[0, :N], p1[1, :N])
    nf3, pn, pc = _tc_node3(nf, aggs, WnA, WnB_p, r1(bn),
                            Wnc1, r1(bnc1), Wnc2, r1(bnc2),
                            Wc1, r1(bc1), Wc2, r1(bc2))

    pe = jnp.concatenate([pea, peb], axis=0)
    e3_64 = jnp.concatenate([e3a64, e3b64], axis=0)
    return (pe.reshape(E), pn.reshape(N), pc, nf3, e3_64)


# speedup vs baseline: 2.1920x; 1.0290x over previous
import functools

import jax
import jax.numpy as jnp
from jax import lax
from jax.experimental import pallas as pl
from jax.experimental.pallas import tpu as pltpu
from jax.experimental.pallas import tpu_sc as plsc

N = 10000
E = 320000
EH = E
D_IN = 128
D_NODE = 128
D_EATTR = 4
D_EDGE = 64
N_TYPES = 17

NC = 2
NS = 16
NW = NC * NS

C = 80
CPW = EH // C // NW
GRP = 5
CS = 40
CPWS = EH // CS // NW
GRPS = 5

NPAD = 10240
RPT = NPAD // NS

BN = 2000
BE = 5000

F32 = jnp.float32


def _dot(a, b):
    return jnp.dot(a, b, preferred_element_type=F32)


def _dot_hi(a, b):
    return _dot(a, b)


def _dot3(a, w):
    return _dot(a, w)


def _mesh():
    return plsc.VectorSubcoreMesh(core_axis_name="c", subcore_axis_name="s")


def _sc_gather(UV1, UV2, src, dst, half):
    @functools.partial(
        pl.kernel,
        out_type=[jax.ShapeDtypeStruct((EH, D_NODE), F32),
                  jax.ShapeDtypeStruct((EH, D_NODE), F32)],
        mesh=_mesh(),
        scratch_types=(
            [pltpu.VMEM((GRP * C,), jnp.int32), pltpu.VMEM((GRP * C,), jnp.int32)]
            + [pltpu.VMEM((C, D_NODE), F32) for _ in range(2 * GRP)]
            + [pltpu.SemaphoreType.DMA, pltpu.SemaphoreType.DMA,
               pltpu.SemaphoreType.DMA]
        ),
    )
    def k(u_hbm, v_hbm, src_hbm, dst_hbm, g1_hbm, g2_hbm, *rest):
        src_v, dst_v = rest[0], rest[1]
        bufs = rest[2:2 + 2 * GRP]
        isem, gsem, wsem = rest[2 + 2 * GRP:]
        wid = lax.axis_index("s") * NC + lax.axis_index("c")
        ebase = half * EH + wid * CPW * C

        def group(g, carry):
            e0 = ebase + g * (GRP * C)
            o0 = e0 - half * EH
            hi0 = pltpu.async_copy(src_hbm.at[pl.ds(e0, GRP * C)], src_v, isem)
            hi1 = pltpu.async_copy(dst_hbm.at[pl.ds(e0, GRP * C)], dst_v, isem)
            hi0.wait()
            hi1.wait()
            hs = []
            for b in range(GRP):
                iu = src_v.at[pl.ds(b * C, C)]
                iv = dst_v.at[pl.ds(b * C, C)]
                hs.append(pltpu.async_copy(u_hbm.at[iu], bufs[2 * b], gsem))
                hs.append(pltpu.async_copy(v_hbm.at[iv], bufs[2 * b + 1], gsem))
            for h in hs:
                h.wait()
            ws = []
            for b in range(GRP):
                row0 = o0 + b * C
                ws.append(pltpu.async_copy(bufs[2 * b], g1_hbm.at[pl.ds(row0, C)], wsem))
                ws.append(pltpu.async_copy(bufs[2 * b + 1], g2_hbm.at[pl.ds(row0, C)], wsem))
            for h in ws:
                h.wait()
            return carry

        lax.fori_loop(0, CPW // GRP, group, 0)

    return k(UV1, UV2, src, dst)


def _sc_scatter(e_feat, dst, zeros_init, half):
    @functools.partial(
        pl.kernel,
        out_type=jax.ShapeDtypeStruct((NC, NPAD, D_NODE), F32),
        mesh=_mesh(),
        scratch_types=(
            [pltpu.VMEM((CS,), jnp.int32) for _ in range(GRPS)]
            + [pltpu.VMEM((GRPS * CS, D_NODE), F32),
               pltpu.VMEM_SHARED((NPAD, D_NODE), F32),
               pltpu.SemaphoreType.DMA, pltpu.SemaphoreType.DMA]
        ),
    )
    def k(e_hbm, dst_hbm, z_hbm, out_hbm, *rest):
        idxbufs = rest[:GRPS]
        dbuf, acc_sh = rest[GRPS], rest[GRPS + 1]
        isem, ssem = rest[GRPS + 2], rest[GRPS + 3]
        c = lax.axis_index("c")
        s = lax.axis_index("s")
        wid = s * NC + c
        obase = wid * CPWS * CS
        ibase = half * EH + obase
        pltpu.sync_copy(z_hbm.at[pl.ds(s * RPT, RPT)], acc_sh.at[pl.ds(s * RPT, RPT)])
        plsc.subcore_barrier()

        def group(g, carry):
            o0 = obase + g * (GRPS * CS)
            i0 = ibase + g * (GRPS * CS)
            his = [pltpu.async_copy(dst_hbm.at[pl.ds(i0 + b * CS, CS)], idxbufs[b], isem)
                   for b in range(GRPS)]
            pltpu.sync_copy(e_hbm.at[pl.ds(o0, GRPS * CS)], dbuf)
            for h in his:
                h.wait()
            hs = []
            for b in range(GRPS):
                hs.append(pltpu.async_copy(
                    dbuf.at[pl.ds(b * CS, CS)], acc_sh.at[idxbufs[b]], ssem,
                    add=True))
            for h in hs:
                h.wait()
            return carry

        lax.fori_loop(0, CPWS // GRPS, group, 0)
        plsc.subcore_barrier()
        pltpu.sync_copy(acc_sh.at[pl.ds(s * RPT, RPT)],
                        out_hbm.at[c, pl.ds(s * RPT, RPT)])

    return k(e_feat, dst, zeros_init)


def _full(shape):
    nd = len(shape)
    return pl.BlockSpec(shape, lambda i, _n=nd: (0,) * _n)


def _tc_embed(x, nt, Wst, bst, W1sd, W1ds):
    def body(x_ref, nt_ref, wst_ref, bst_ref, w1s_ref, w1d_ref,
             nf_ref, u_ref, v_ref):
        xb = x_ref[...]
        ntb = nt_ref[...]
        acc = jnp.zeros((BN, D_NODE), F32)
        for t in range(N_TYPES):
            yt = _dot_hi(xb, wst_ref[t]) + bst_ref[t][None]
            acc = acc + jnp.where(ntb == t, yt, 0.0)
        nf_ref[...] = acc
        u_ref[...] = _dot_hi(acc, w1s_ref[...])
        v_ref[...] = _dot_hi(acc, w1d_ref[...])

    return pl.pallas_call(
        body,
        grid=(N // BN,),
        in_specs=[
            pl.BlockSpec((BN, D_IN), lambda i: (i, 0)),
            pl.BlockSpec((BN, 1), lambda i: (i, 0)),
            _full((N_TYPES, D_IN, D_NODE)),
            _full((N_TYPES, D_NODE)),
            _full((D_NODE, D_NODE)),
            _full((D_NODE, D_NODE)),
        ],
        out_specs=[
            pl.BlockSpec((BN, D_NODE), lambda i: (i, 0)),
            pl.BlockSpec((BN, D_NODE), lambda i: (i, 0)),
            pl.BlockSpec((BN, D_NODE), lambda i: (i, 0)),
        ],
        out_shape=[
            jax.ShapeDtypeStruct((N, D_NODE), F32),
            jax.ShapeDtypeStruct((N, D_NODE), F32),
            jax.ShapeDtypeStruct((N, D_NODE), F32),
        ],
    )(x, nt, Wst, bst, W1sd, W1ds)


def _tc_edge1(G1, G2, ea, We, be, W1e_p, b1_p, W2_pp, b2_p):
    def body(g1_ref, g2_ref, ea_ref, we_ref, be_ref, w1e_ref, b1_ref,
             w2_ref, b2_ref, e_ref):
        ef = _dot(ea_ref[...], we_ref[...]) + be_ref[...]
        p = _dot(ef, w1e_ref[...])
        h = jnp.maximum(g1_ref[...] + g2_ref[...] + p + b1_ref[...], 0.0)
        e_ref[...] = _dot(h, w2_ref[...]) + b2_ref[...]

    return pl.pallas_call(
        body,
        grid=(EH // BE,),
        in_specs=[
            pl.BlockSpec((BE, D_NODE), lambda i: (i, 0)),
            pl.BlockSpec((BE, D_NODE), lambda i: (i, 0)),
            pl.BlockSpec((BE, D_EATTR), lambda i: (i, 0)),
            _full((D_EATTR, D_EDGE)),
            _full((1, D_EDGE)),
            _full((D_EDGE, D_NODE)),
            _full((1, D_NODE)),
            _full((D_NODE, D_NODE)),
            _full((1, D_NODE)),
        ],
        out_specs=pl.BlockSpec((BE, D_NODE), lambda i: (i, 0)),
        out_shape=jax.ShapeDtypeStruct((EH, D_NODE), F32),
    )(G1, G2, ea, We, be, W1e_p, b1_p, W2_pp, b2_p)


def _tc_edge2(G1, G2, ep, W1e_pp, b1_p, W2_pp, b2_p, Wec1_p, bec1, Wec2, bec2):
    def body(g1_ref, g2_ref, ep_ref, w1e_ref, b1_ref, w2_ref, b2_ref,
             wec1_ref, bec1_ref, wec2_ref, bec2_ref, e_ref, pe_ref):
        p = _dot3(ep_ref[...], w1e_ref[...])
        h = jnp.maximum(g1_ref[...] + g2_ref[...] + p + b1_ref[...], 0.0)
        e_new = _dot3(h, w2_ref[...]) + b2_ref[...]
        e_ref[...] = e_new
        t = jnp.maximum(_dot(e_new, wec1_ref[...]) + bec1_ref[...], 0.0)
        pe_ref[...] = _dot(t, wec2_ref[...]) + bec2_ref[...]

    return pl.pallas_call(
        body,
        grid=(EH // BE,),
        in_specs=[
            pl.BlockSpec((BE, D_NODE), lambda i: (i, 0)),
            pl.BlockSpec((BE, D_NODE), lambda i: (i, 0)),
            pl.BlockSpec((BE, D_NODE), lambda i: (i, 0)),
            _full((D_NODE, D_NODE)),
            _full((1, D_NODE)),
            _full((D_NODE, D_NODE)),
            _full((1, D_NODE)),
            _full((D_NODE, 32)),
            _full((1, 32)),
            _full((32, 1)),
            _full((1, 1)),
        ],
        out_specs=[
            pl.BlockSpec((BE, D_NODE), lambda i: (i, 0)),
            pl.BlockSpec((BE, 1), lambda i: (i, 0)),
        ],
        out_shape=[
            jax.ShapeDtypeStruct((EH, D_NODE), F32),
            jax.ShapeDtypeStruct((EH, 1), F32),
        ],
    )(G1, G2, ep, W1e_pp, b1_p, W2_pp, b2_p, Wec1_p, bec1, Wec2, bec2)


def _tc_edge3(G1, G2, ep, W1e_pp, b1_p, W2_pp, b2_p):
    def body(g1_ref, g2_ref, ep_ref, w1e_ref, b1_ref, w2_ref, b2_ref,
             e_ref, e64_ref):
        p = _dot3(ep_ref[...], w1e_ref[...])
        h = jnp.maximum(g1_ref[...] + g2_ref[...] + p + b1_ref[...], 0.0)
        e_new = _dot3(h, w2_ref[...]) + b2_ref[...]
        e_ref[...] = e_new
        e64_ref[...] = e_new[:, :D_EDGE]

    return pl.pallas_call(
        body,
        grid=(EH // BE,),
        in_specs=[
            pl.BlockSpec((BE, D_NODE), lambda i: (i, 0)),
            pl.BlockSpec((BE, D_NODE), lambda i: (i, 0)),
            pl.BlockSpec((BE, D_NODE), lambda i: (i, 0)),
            _full((D_NODE, D_NODE)),
            _full((1, D_NODE)),
            _full((D_NODE, D_NODE)),
            _full((1, D_NODE)),
        ],
        out_specs=[
            pl.BlockSpec((BE, D_NODE), lambda i: (i, 0)),
            pl.BlockSpec((BE, D_EDGE), lambda i: (i, 0)),
        ],
        out_shape=[
            jax.ShapeDtypeStruct((EH, D_NODE), F32),
            jax.ShapeDtypeStruct((EH, D_EDGE), F32),
        ],
    )(G1, G2, ep, W1e_pp, b1_p, W2_pp, b2_p)


def _tc_node(nf, aggs, WnA, WnB_p, bn, W1sd, W1ds):
    def body(nf_ref, a0_ref, a1_ref,
             wna_ref, wnb_ref, bn_ref, w1s_ref, w1d_ref,
             o_ref, u_ref, v_ref):
        agg = a0_ref[...] + a1_ref[...]
        nn = _dot_hi(nf_ref[...], wna_ref[...]) + _dot_hi(agg, wnb_ref[...]) + bn_ref[...]
        o_ref[...] = nn
        u_ref[...] = _dot_hi(nn, w1s_ref[...])
        v_ref[...] = _dot_hi(nn, w1d_ref[...])

    aspec = pl.BlockSpec((BN, D_NODE), lambda i: (i, 0))
    return pl.pallas_call(
        body,
        grid=(N // BN,),
        in_specs=[aspec, aspec, aspec,
                  _full((D_NODE, D_NODE)),
                  _full((D_NODE, D_NODE)),
                  _full((1, D_NODE)),
                  _full((D_NODE, D_NODE)),
                  _full((D_NODE, D_NODE))],
        out_specs=[aspec, aspec, aspec],
        out_shape=[
            jax.ShapeDtypeStruct((N, D_NODE), F32),
            jax.ShapeDtypeStruct((N, D_NODE), F32),
            jax.ShapeDtypeStruct((N, D_NODE), F32),
        ],
    )(nf, aggs[0], aggs[1], WnA, WnB_p, bn, W1sd, W1ds)


def _tc_node3(nf, aggs, WnA, WnB_p, bn, Wnc1, bnc1, Wnc2, bnc2, Wc1, bc1, Wc2, bc2):
    def body(nf_ref, a0_ref, a1_ref, wna_ref, wnb_ref, bn_ref,
             wnc1_ref, bnc1_ref, wnc2_ref, bnc2_ref,
             wc1_ref, bc1_ref, wc2_ref, bc2_ref,
             o_ref, pn_ref, pc_ref):
        agg = a0_ref[...] + a1_ref[...]
        nn = _dot_hi(nf_ref[...], wna_ref[...]) + _dot_hi(agg, wnb_ref[...]) + bn_ref[...]
        o_ref[...] = nn
        t1 = jnp.maximum(_dot_hi(nn, wnc1_ref[...]) + bnc1_ref[...], 0.0)
        pn_ref[...] = _dot_hi(t1, wnc2_ref[...]) + bnc2_ref[...]
        t2 = jnp.maximum(_dot_hi(nn, wc1_ref[...]) + bc1_ref[...], 0.0)
        pc_ref[...] = _dot_hi(t2, wc2_ref[...]) + bc2_ref[...]

    aspec = pl.BlockSpec((BN, D_NODE), lambda i: (i, 0))
    return pl.pallas_call(
        body,
        grid=(N // BN,),
        in_specs=[aspec, aspec, aspec,
                  _full((D_NODE, D_NODE)),
                  _full((D_NODE, D_NODE)),
                  _full((1, D_NODE)),
                  _full((D_NODE, 32)),
                  _full((1, 32)),
                  _full((32, 1)),
                  _full((1, 1)),
                  _full((D_NODE, 32)),
                  _full((1, 32)),
                  _full((32, 8)),
                  _full((1, 8))],
        out_specs=[aspec,
                   pl.BlockSpec((BN, 1), lambda i: (i, 0)),
                   pl.BlockSpec((BN, 8), lambda i: (i, 0))],
        out_shape=[
            jax.ShapeDtypeStruct((N, D_NODE), F32),
            jax.ShapeDtypeStruct((N, 1), F32),
            jax.ShapeDtypeStruct((N, 8), F32),
        ],
    )(nf, aggs[0], aggs[1], WnA, WnB_p, bn,
      Wnc1, bnc1, Wnc2, bnc2, Wc1, bc1, Wc2, bc2)


def kernel(x, edge_attr, edge_index, node_types, params):
    src = edge_index[0]
    dst = edge_index[1]
    nt = node_types.reshape(N, 1)

    Wst = jnp.stack([p[0][0] for p in params['node_mlps']])
    bst = jnp.stack([p[0][1] for p in params['node_mlps']])
    We, be = params['edge_emb'][0]
    (W1, b1), (W2, b2) = params['mp_edge']
    Wn, bn = params['mp_node'][0]
    (Wec1, bec1), (Wec2, bec2) = params['edge_cls']
    (Wnc1, bnc1), (Wnc2, bnc2) = params['node_cls']
    (Wc1, bc1), (Wc2, bc2) = params['cls']

    W1s, W1d, W1e = W1[:D_NODE], W1[D_NODE:2 * D_NODE], W1[2 * D_NODE:]
    WnA, WnB = Wn[:D_NODE], Wn[D_NODE:]
    r1 = lambda v: v.reshape(1, -1)

    zc = jnp.zeros((D_EDGE, D_EDGE), F32)
    zr = jnp.zeros((D_EDGE,), F32)
    W1sd = jnp.concatenate([W1s, W1d], axis=1)
    W1ds = jnp.concatenate([W1d, W1s], axis=1)
    W1e_p = jnp.concatenate([W1e, zc], axis=1)
    W1e_pp = jnp.concatenate([W1e_p, jnp.zeros((D_EDGE, D_NODE), F32)], axis=0)
    W2_pp = jnp.concatenate(
        [jnp.concatenate([W2, zc], axis=1),
         jnp.zeros((D_EDGE, D_NODE), F32)], axis=0)
    b1_p = jnp.concatenate([b1, zr])
    b2_p = jnp.concatenate([b2, zr])
    WnB_p = jnp.concatenate([WnB, jnp.zeros((D_EDGE, D_NODE), F32)], axis=0)
    Wec1_p = jnp.concatenate([Wec1, jnp.zeros((D_EDGE, 32), F32)], axis=0)

    zeros_init = jnp.zeros((NPAD, D_NODE), F32)
    ea0, ea1 = edge_attr[:EH], edge_attr[EH:]

    nf, U, V = _tc_embed(x, nt, Wst, bst, W1sd, W1ds)

    G1, G2 = _sc_gather(U, V, src, dst, 0)
    e1 = _tc_edge1(G1, G2, edge_attr, We, r1(be), W1e_p, r1(b1_p), W2_pp, r1(b2_p))
    p = _sc_scatter(e1, dst, zeros_init, 0)
    aggs = (p[0, :N], p[1, :N])
    nf, U, V = _tc_node(nf, aggs, WnA, WnB_p, r1(bn), W1sd, W1ds)

    G1, G2 = _sc_gather(U, V, src, dst, 0)
    e2, pe = _tc_edge2(G1, G2, e1, W1e_pp, r1(b1_p), W2_pp, r1(b2_p),
                       Wec1_p, r1(bec1), Wec2, r1(bec2))
    p = _sc_scatter(e2, dst, zeros_init, 0)
    aggs = (p[0, :N], p[1, :N])
    nf, U, V = _tc_node(nf, aggs, WnA, WnB_p, r1(bn), W1sd, W1ds)

    G1, G2 = _sc_gather(U, V, src, dst, 0)
    e3, e3_64 = _tc_edge3(G1, G2, e2, W1e_pp, r1(b1_p), W2_pp, r1(b2_p))
    p = _sc_scatter(e3, dst, zeros_init, 0)
    aggs = (p[0, :N], p[1, :N])
    nf3, pn, pc = _tc_node3(nf, aggs, WnA, WnB_p, r1(bn),
                            Wnc1, r1(bnc1), Wnc2, r1(bnc2),
                            Wc1, r1(bc1), Wc2, r1(bc2))

    return (pe.reshape(E), pn.reshape(N), pc, nf3, e3_64)
